# baseline (speedup 1.0000x reference)
import functools

import jax
import jax.numpy as jnp
from jax import lax
from jax.experimental import pallas as pl
from jax.experimental.pallas import tpu as pltpu
from jax.experimental.pallas import tpu_sc as plsc

_NC = 2
_NS = 16
_NW = _NC * _NS
_CH = 128
_ROW_BLK = 1024


def _matmul_body(x_ref, w_ref, o_ref):
    o_ref[...] = jnp.dot(x_ref[...], w_ref[...], preferred_element_type=jnp.float32, precision=lax.Precision.HIGHEST)


def _mlp_mid_body(t_ref, p0_ref, p1_ref, b1a_ref, w1b_ref, b1b_ref, w2a_ref, o_ref):
    a = jnp.maximum(t_ref[...] + p0_ref[...] + p1_ref[...] + b1a_ref[...], 0.0)
    h1 = jnp.dot(a, w1b_ref[...], preferred_element_type=jnp.float32, precision=lax.Precision.HIGHEST) + b1b_ref[...]
    o_ref[...] = jnp.dot(h1, w2a_ref[...], preferred_element_type=jnp.float32, precision=lax.Precision.HIGHEST)


def _mlp_out_body(u_ref, q0_ref, q1_ref, b2a_ref, w2b_ref, b2b_ref, wo_ref, bo_ref,
                  o_ref):
    a = jnp.maximum(u_ref[...] + q0_ref[...] + q1_ref[...] + b2a_ref[...], 0.0)
    h2 = jnp.dot(a, w2b_ref[...], preferred_element_type=jnp.float32, precision=lax.Precision.HIGHEST) + b2b_ref[...]
    o_ref[...] = jnp.sum(h2 * wo_ref[...], axis=1, keepdims=True) + bo_ref[...]


_NU = 2


@functools.lru_cache(maxsize=None)
def _make_sc_agg(np_, h, nchw):
    rpt = np_ // _NS
    ng = nchw // _NU
    nzc = rpt // _CH
    assert nchw % _NU == 0
    mesh = plsc.VectorSubcoreMesh(core_axis_name="c", subcore_axis_name="s")

    @functools.partial(
        pl.kernel,
        out_type=jax.ShapeDtypeStruct((_NC, np_, h), jnp.float32),
        mesh=mesh,
        scratch_types=[
            pltpu.VMEM((nchw, _CH), jnp.int32),
            pltpu.VMEM((nchw, _CH), jnp.int32),
            pltpu.VMEM((_NU, _CH, h), jnp.float32),
            pltpu.VMEM_SHARED((np_, h), jnp.float32),
            pltpu.VMEM_SHARED((np_, h), jnp.float32),
            [pltpu.SemaphoreType.DMA] * _NU,
            [pltpu.SemaphoreType.DMA] * _NU,
        ],
        compiler_params=pltpu.CompilerParams(use_tc_tiling_on_sc=False),
    )
    def agg(t_hbm, src_hbm, dst_hbm, zero_hbm, out_hbm, src_v, dst_v, rows_v,
            acc_sh, t_sh, sg, ss):
        c = lax.axis_index("c")
        s = lax.axis_index("s")
        wid = c * _NS + s
        base = s * rpt
        pltpu.sync_copy(t_hbm.at[pl.ds(base, rpt)], t_sh.at[pl.ds(base, rpt)])
        pltpu.sync_copy(zero_hbm, rows_v.at[0])
        for k in range(nzc):
            pltpu.sync_copy(rows_v.at[0], acc_sh.at[pl.ds(base + k * _CH, _CH)])
        pltpu.sync_copy(src_hbm.at[wid], src_v)
        pltpu.sync_copy(dst_hbm.at[wid], dst_v)
        plsc.subcore_barrier()

        def scat_wait(j0, k):
            pltpu.make_async_copy(rows_v.at[k], acc_sh.at[dst_v.at[j0 + k]],
                                  ss[k]).wait()

        def group(g, carry):
            j0 = g * _NU

            @pl.when(g > 0)
            def _():
                for k in range(_NU):
                    scat_wait(j0 - _NU, k)

            gat = [pltpu.async_copy(t_sh.at[src_v.at[j0 + k]], rows_v.at[k],
                                    sg[k]) for k in range(_NU)]
            for k in range(_NU):
                gat[k].wait()
                pltpu.async_copy(rows_v.at[k], acc_sh.at[dst_v.at[j0 + k]],
                                 ss[k], add=True)
            return carry

        lax.fori_loop(0, ng, group, 0)
        for k in range(_NU):
            scat_wait((ng - 1) * _NU, k)
        plsc.subcore_barrier()
        for k in range(nzc):
            pltpu.sync_copy(acc_sh.at[pl.ds(base + k * _CH, _CH)], rows_v.at[0])
            pltpu.sync_copy(rows_v.at[0], out_hbm.at[c, pl.ds(base + k * _CH, _CH)])

    return agg


def _blk(shp):
    return pl.BlockSpec(shp, lambda i: (i, 0))


def _whole(shp):
    return pl.BlockSpec(shp, lambda i: (0, 0))


def kernel(x, edge_index, W1a, b1a, W1b, b1b, W2a, b2a, W2b, b2b, Wo, bo):
    n, d = x.shape
    h = W1a.shape[1]
    e = edge_index.shape[1]
    np_ = ((n + 1 + _ROW_BLK - 1) // _ROW_BLK) * _ROW_BLK
    nblk = np_ // _ROW_BLK
    nchw = -(-e // (_NW * _CH * _NU)) * _NU
    e_pad = _NW * _CH * nchw

    fill = jnp.full((e_pad - e,), n, jnp.int32)
    src = jnp.concatenate([edge_index[0], fill]).reshape(_NW, nchw, _CH)
    dst = jnp.concatenate([edge_index[1], fill]).reshape(_NW, nchw, _CH)
    x_pad = jnp.pad(x, ((0, np_ - n), (0, 0)))
    zeros = jnp.zeros((_CH, h), jnp.float32)

    b1a_r, b1b_r, b2a_r, b2b_r = (v.reshape(1, h) for v in (b1a, b1b, b2a, b2b))
    wo_r = Wo.reshape(1, h)
    bo_r = bo.reshape(1, 1)

    t = pl.pallas_call(
        _matmul_body,
        grid=(nblk,),
        in_specs=[_blk((_ROW_BLK, d)), _whole((d, h))],
        out_specs=_blk((_ROW_BLK, h)),
        out_shape=jax.ShapeDtypeStruct((np_, h), jnp.float32),
    )(x_pad, W1a)

    sc_agg = _make_sc_agg(np_, h, nchw)

    p = sc_agg(t, src, dst, zeros)

    u = pl.pallas_call(
        _mlp_mid_body,
        grid=(nblk,),
        in_specs=[_blk((_ROW_BLK, h)), _blk((_ROW_BLK, h)), _blk((_ROW_BLK, h)),
                  _whole((1, h)), _whole((h, h)), _whole((1, h)), _whole((h, h))],
        out_specs=_blk((_ROW_BLK, h)),
        out_shape=jax.ShapeDtypeStruct((np_, h), jnp.float32),
    )(t, p[0], p[1], b1a_r, W1b, b1b_r, W2a)

    q = sc_agg(u, src, dst, zeros)

    res = pl.pallas_call(
        _mlp_out_body,
        grid=(nblk,),
        in_specs=[_blk((_ROW_BLK, h)), _blk((_ROW_BLK, h)), _blk((_ROW_BLK, h)),
                  _whole((1, h)), _whole((h, h)), _whole((1, h)), _whole((1, h)),
                  _whole((1, 1))],
        out_specs=_blk((_ROW_BLK, 1)),
        out_shape=jax.ShapeDtypeStruct((np_, 1), jnp.float32),
    )(u, q[0], q[1], b2a_r, W2b, b2b_r, wo_r, bo_r)

    return res[:n, 0]


# speedup vs baseline: 1.1045x; 1.1045x over previous
import functools

import jax
import jax.numpy as jnp
from jax import lax
from jax.experimental import pallas as pl
from jax.experimental.pallas import tpu as pltpu
from jax.experimental.pallas import tpu_sc as plsc

_NC = 2
_NS = 16
_NW = _NC * _NS
_CH = 128
_ROW_BLK = 1024


def _matmul_body(x_ref, w_ref, o_ref):
    o_ref[...] = jnp.dot(x_ref[...], w_ref[...], preferred_element_type=jnp.float32, precision=lax.Precision.HIGHEST)


def _mlp_mid_body(t_ref, p_ref, b1a_ref, w1b_ref, b1b_ref, w2a_ref, o_ref):
    w_mid = jnp.dot(w1b_ref[...], w2a_ref[...],
                    preferred_element_type=jnp.float32,
                    precision=lax.Precision.HIGHEST)
    b_mid = jnp.dot(b1b_ref[...], w2a_ref[...],
                    preferred_element_type=jnp.float32,
                    precision=lax.Precision.HIGHEST)
    a = jnp.maximum(t_ref[...] + p_ref[0] + p_ref[1] + b1a_ref[...], 0.0)
    o_ref[...] = jnp.dot(a, w_mid, preferred_element_type=jnp.float32,
                         precision=lax.Precision.HIGHEST) + b_mid


def _mlp_out_body(u_ref, q_ref, b2a_ref, w2b_ref, b2b_ref, wo_ref, bo_ref,
                  o_ref):
    wv = jnp.dot(w2b_ref[...], wo_ref[...], preferred_element_type=jnp.float32,
                 precision=lax.Precision.HIGHEST)
    bv = jnp.dot(b2b_ref[...], wo_ref[...], preferred_element_type=jnp.float32,
                 precision=lax.Precision.HIGHEST) + bo_ref[...]
    a = jnp.maximum(u_ref[...] + q_ref[0] + q_ref[1] + b2a_ref[...], 0.0)
    o_ref[...] = jnp.sum(a * wv[:, 0], axis=1, keepdims=True) + bv


_NU = 2


@functools.lru_cache(maxsize=None)
def _make_sc_agg(np_, h, nchw):
    rpt = np_ // _NS
    ng = nchw // _NU
    nzc = rpt // _CH
    assert nchw % _NU == 0
    mesh = plsc.VectorSubcoreMesh(core_axis_name="c", subcore_axis_name="s")

    @functools.partial(
        pl.kernel,
        out_type=jax.ShapeDtypeStruct((_NC, np_, h), jnp.float32),
        mesh=mesh,
        scratch_types=[
            pltpu.VMEM((nchw, _CH), jnp.int32),
            pltpu.VMEM((nchw, _CH), jnp.int32),
            pltpu.VMEM((_NU, _CH, h), jnp.float32),
            pltpu.VMEM_SHARED((np_, h), jnp.float32),
            pltpu.VMEM_SHARED((np_, h), jnp.float32),
            [pltpu.SemaphoreType.DMA] * _NU,
            [pltpu.SemaphoreType.DMA] * _NU,
        ],
        compiler_params=pltpu.CompilerParams(use_tc_tiling_on_sc=False),
    )
    def agg(t_hbm, edges_hbm, zero_hbm, out_hbm, src_v, dst_v, rows_v,
            acc_sh, t_sh, sg, ss):
        c = lax.axis_index("c")
        s = lax.axis_index("s")
        wid = c * _NS + s
        base = s * rpt
        pltpu.sync_copy(t_hbm.at[pl.ds(base, rpt)], t_sh.at[pl.ds(base, rpt)])
        pltpu.sync_copy(zero_hbm, rows_v.at[0])
        for k in range(nzc):
            pltpu.sync_copy(rows_v.at[0], acc_sh.at[pl.ds(base + k * _CH, _CH)])
        pltpu.sync_copy(edges_hbm.at[0, wid], src_v)
        pltpu.sync_copy(edges_hbm.at[1, wid], dst_v)
        plsc.subcore_barrier()

        def group(g, carry):
            j0 = g * _NU
            gat = [pltpu.async_copy(t_sh.at[src_v.at[j0 + k]], rows_v.at[k],
                                    sg[k]) for k in range(_NU)]
            sca = []
            for k in range(_NU):
                gat[k].wait()
                sca.append(pltpu.async_copy(rows_v.at[k],
                                            acc_sh.at[dst_v.at[j0 + k]],
                                            ss[k], add=True))
            for d in sca:
                d.wait()
            return carry

        lax.fori_loop(0, ng, group, 0)
        plsc.subcore_barrier()
        for k in range(nzc):
            pltpu.sync_copy(acc_sh.at[pl.ds(base + k * _CH, _CH)], rows_v.at[0])
            pltpu.sync_copy(rows_v.at[0], out_hbm.at[c, pl.ds(base + k * _CH, _CH)])

    return agg


def _blk(shp):
    return pl.BlockSpec(shp, lambda i: (i, 0))


def _whole(shp):
    return pl.BlockSpec(shp, lambda i: (0, 0))


def kernel(x, edge_index, W1a, b1a, W1b, b1b, W2a, b2a, W2b, b2b, Wo, bo):
    n, d = x.shape
    h = W1a.shape[1]
    e = edge_index.shape[1]
    np_ = ((n + 1 + _ROW_BLK - 1) // _ROW_BLK) * _ROW_BLK
    nblk = np_ // _ROW_BLK
    nchw = -(-e // (_NW * _CH * _NU)) * _NU
    e_pad = _NW * _CH * nchw

    edges = jnp.pad(edge_index, ((0, 0), (0, e_pad - e)),
                    constant_values=n).reshape(2, _NW, nchw, _CH)
    x_pad = jnp.pad(x, ((0, np_ - n), (0, 0)))
    zeros = jnp.zeros((_CH, h), jnp.float32)

    b1a_r, b1b_r, b2a_r, b2b_r = (v.reshape(1, h) for v in (b1a, b1b, b2a, b2b))
    bo_r = bo.reshape(1, 1)

    t = pl.pallas_call(
        _matmul_body,
        grid=(nblk,),
        in_specs=[_blk((_ROW_BLK, d)), _whole((d, h))],
        out_specs=_blk((_ROW_BLK, h)),
        out_shape=jax.ShapeDtypeStruct((np_, h), jnp.float32),
    )(x_pad, W1a)

    sc_agg = _make_sc_agg(np_, h, nchw)

    pblk = pl.BlockSpec((2, _ROW_BLK, h), lambda i: (0, i, 0))

    p = sc_agg(t, edges, zeros)

    u = pl.pallas_call(
        _mlp_mid_body,
        grid=(nblk,),
        in_specs=[_blk((_ROW_BLK, h)), pblk,
                  _whole((1, h)), _whole((h, h)), _whole((1, h)), _whole((h, h))],
        out_specs=_blk((_ROW_BLK, h)),
        out_shape=jax.ShapeDtypeStruct((np_, h), jnp.float32),
    )(t, p, b1a_r, W1b, b1b_r, W2a)

    q = sc_agg(u, edges, zeros)

    res = pl.pallas_call(
        _mlp_out_body,
        grid=(nblk,),
        in_specs=[_blk((_ROW_BLK, h)), pblk,
                  _whole((1, h)), _whole((h, h)), _whole((1, h)),
                  _whole((h, 1)), _whole((1, 1))],
        out_specs=_blk((_ROW_BLK, 1)),
        out_shape=jax.ShapeDtypeStruct((np_, 1), jnp.float32),
    )(u, q, b2a_r, W2b, b2b_r, Wo, bo_r)

    return res[:n, 0]


# speedup vs baseline: 1.2235x; 1.1078x over previous
import functools

import jax
import jax.numpy as jnp
from jax import lax
from jax.experimental import pallas as pl
from jax.experimental.pallas import tpu as pltpu
from jax.experimental.pallas import tpu_sc as plsc

_NC = 2
_NS = 16
_NW = _NC * _NS
_CH = 128
_ROW_BLK = 1024


def _blockdiag(w):
    h_in, h_out = w.shape
    z = jnp.zeros((h_in, h_out), jnp.float32)
    top = jnp.concatenate([w, z], axis=1)
    bot = jnp.concatenate([z, w], axis=1)
    return jnp.concatenate([top, bot], axis=0)


def _matmul_body(x2_ref, w_ref, o_ref):
    w2 = _blockdiag(w_ref[...])
    o_ref[...] = jnp.dot(x2_ref[...], w2, preferred_element_type=jnp.float32,
                         precision=lax.Precision.HIGHEST)


def _mlp_mid_body(t_ref, p_ref, b1a2_ref, w1b_ref, b1b_ref, w2a_ref, o_ref):
    w_mid = jnp.dot(w1b_ref[...], w2a_ref[...],
                    preferred_element_type=jnp.float32,
                    precision=lax.Precision.HIGHEST)
    b_mid = jnp.dot(b1b_ref[...], w2a_ref[...],
                    preferred_element_type=jnp.float32,
                    precision=lax.Precision.HIGHEST)
    b_mid2 = jnp.concatenate([b_mid, b_mid], axis=1)
    a = jnp.maximum(t_ref[...] + p_ref[0] + p_ref[1] + b1a2_ref[...], 0.0)
    o_ref[...] = jnp.dot(a, _blockdiag(w_mid),
                         preferred_element_type=jnp.float32,
                         precision=lax.Precision.HIGHEST) + b_mid2


def _mlp_out_body(u_ref, q_ref, b2a2_ref, w2b_ref, b2b_ref, wo_ref, bo_ref,
                  o_ref):
    h = w2b_ref.shape[0]
    wv = jnp.dot(w2b_ref[...], wo_ref[...], preferred_element_type=jnp.float32,
                 precision=lax.Precision.HIGHEST)
    bv = jnp.dot(b2b_ref[...], wo_ref[...], preferred_element_type=jnp.float32,
                 precision=lax.Precision.HIGHEST) + bo_ref[...]
    a = jnp.maximum(u_ref[...] + q_ref[0] + q_ref[1] + b2a2_ref[...], 0.0)
    s0 = jnp.sum(a[:, :h] * wv[:, 0], axis=1, keepdims=True)
    s1 = jnp.sum(a[:, h:] * wv[:, 0], axis=1, keepdims=True)
    o_ref[...] = jnp.concatenate([s0, s1], axis=1) + bv


_NU = 2


@functools.lru_cache(maxsize=None)
def _make_sc_agg(np_, h, nchw):
    rpt = np_ // _NS
    ng = nchw // _NU
    nzc = rpt // _CH
    assert nchw % _NU == 0
    mesh = plsc.VectorSubcoreMesh(core_axis_name="c", subcore_axis_name="s")

    @functools.partial(
        pl.kernel,
        out_type=jax.ShapeDtypeStruct((_NC, np_, h), jnp.float32),
        mesh=mesh,
        scratch_types=[
            pltpu.VMEM((nchw, _CH), jnp.int32),
            pltpu.VMEM((nchw, _CH), jnp.int32),
            pltpu.VMEM((_NU, _CH, h), jnp.float32),
            pltpu.VMEM_SHARED((np_, h), jnp.float32),
            pltpu.VMEM_SHARED((np_, h), jnp.float32),
            [pltpu.SemaphoreType.DMA] * _NU,
            [pltpu.SemaphoreType.DMA] * _NU,
        ],
        compiler_params=pltpu.CompilerParams(use_tc_tiling_on_sc=False),
    )
    def agg(t_hbm, edges_hbm, zero_hbm, out_hbm, src_v, dst_v, rows_v,
            acc_sh, t_sh, sg, ss):
        c = lax.axis_index("c")
        s = lax.axis_index("s")
        wid = c * _NS + s
        base = s * rpt
        pltpu.sync_copy(t_hbm.at[pl.ds(base, rpt)], t_sh.at[pl.ds(base, rpt)])
        pltpu.sync_copy(zero_hbm, rows_v.at[0])
        for k in range(nzc):
            pltpu.sync_copy(rows_v.at[0], acc_sh.at[pl.ds(base + k * _CH, _CH)])
        pltpu.sync_copy(edges_hbm.at[0, wid], src_v)
        pltpu.sync_copy(edges_hbm.at[1, wid], dst_v)
        plsc.subcore_barrier()

        def group(g, carry):
            j0 = g * _NU
            gat = [pltpu.async_copy(t_sh.at[src_v.at[j0 + k]], rows_v.at[k],
                                    sg[k]) for k in range(_NU)]
            sca = []
            for k in range(_NU):
                gat[k].wait()
                sca.append(pltpu.async_copy(rows_v.at[k],
                                            acc_sh.at[dst_v.at[j0 + k]],
                                            ss[k], add=True))
            for d in sca:
                d.wait()
            return carry

        lax.fori_loop(0, ng, group, 0)
        plsc.subcore_barrier()
        for k in range(nzc):
            pltpu.sync_copy(acc_sh.at[pl.ds(base + k * _CH, _CH)], rows_v.at[0])
            pltpu.sync_copy(rows_v.at[0], out_hbm.at[c, pl.ds(base + k * _CH, _CH)])

    return agg


def _blk(shp):
    return pl.BlockSpec(shp, lambda i: (i, 0))


def _whole(shp):
    return pl.BlockSpec(shp, lambda i: (0, 0))


def kernel(x, edge_index, W1a, b1a, W1b, b1b, W2a, b2a, W2b, b2b, Wo, bo):
    n, d = x.shape
    h = W1a.shape[1]
    e = edge_index.shape[1]
    np_ = ((n + 1 + _ROW_BLK - 1) // _ROW_BLK) * _ROW_BLK
    nblk = np_ // _ROW_BLK
    nchw = -(-e // (_NW * _CH * _NU)) * _NU
    e_pad = _NW * _CH * nchw

    npk = np_ // 2
    w2h = 2 * h
    pblk2 = npk // nblk

    edges = jnp.pad(edge_index, ((0, 0), (0, e_pad - e)),
                    constant_values=n).reshape(2, _NW, nchw, _CH)
    x2 = jnp.pad(x, ((0, np_ - n), (0, 0))).reshape(npk, 2 * d)
    zeros = jnp.zeros((_CH, h), jnp.float32)

    b1a2, b2a2 = (jnp.concatenate([v, v]).reshape(1, w2h) for v in (b1a, b2a))
    b1b_r, b2b_r = b1b.reshape(1, h), b2b.reshape(1, h)
    bo_r = bo.reshape(1, 1)

    t2 = pl.pallas_call(
        _matmul_body,
        grid=(nblk,),
        in_specs=[_blk((pblk2, 2 * d)), _whole((d, h))],
        out_specs=_blk((pblk2, w2h)),
        out_shape=jax.ShapeDtypeStruct((npk, w2h), jnp.float32),
    )(x2, W1a)

    sc_agg = _make_sc_agg(np_, h, nchw)

    pspec = pl.BlockSpec((2, pblk2, w2h), lambda i: (0, i, 0))

    p = sc_agg(t2.reshape(np_, h), edges, zeros)
    p2 = p.reshape(2, npk, w2h)

    u2 = pl.pallas_call(
        _mlp_mid_body,
        grid=(nblk,),
        in_specs=[_blk((pblk2, w2h)), pspec,
                  _whole((1, w2h)), _whole((h, h)), _whole((1, h)),
                  _whole((h, h))],
        out_specs=_blk((pblk2, w2h)),
        out_shape=jax.ShapeDtypeStruct((npk, w2h), jnp.float32),
    )(t2, p2, b1a2, W1b, b1b_r, W2a)

    q = sc_agg(u2.reshape(np_, h), edges, zeros)
    q2 = q.reshape(2, npk, w2h)

    res = pl.pallas_call(
        _mlp_out_body,
        grid=(nblk,),
        in_specs=[_blk((pblk2, w2h)), pspec,
                  _whole((1, w2h)), _whole((h, h)), _whole((1, h)),
                  _whole((h, 1)), _whole((1, 1))],
        out_specs=_blk((pblk2, 2)),
        out_shape=jax.ShapeDtypeStruct((npk, 2), jnp.float32),
    )(u2, q2, b2a2, W2b, b2b_r, Wo, bo_r)

    return res.reshape(np_)[:n]


# speedup vs baseline: 1.2296x; 1.0050x over previous
import functools

import jax
import jax.numpy as jnp
from jax import lax
from jax.experimental import pallas as pl
from jax.experimental.pallas import tpu as pltpu
from jax.experimental.pallas import tpu_sc as plsc

_NC = 2
_NS = 16
_NW = _NC * _NS
_CH = 128
_ROW_BLK = 1024


def _blockdiag(w):
    h_in, h_out = w.shape
    z = jnp.zeros((h_in, h_out), jnp.float32)
    top = jnp.concatenate([w, z], axis=1)
    bot = jnp.concatenate([z, w], axis=1)
    return jnp.concatenate([top, bot], axis=0)


def _matmul_body(x2_ref, w_ref, o_ref):
    w2 = _blockdiag(w_ref[...])
    o_ref[...] = jnp.dot(x2_ref[...], w2, preferred_element_type=jnp.float32,
                         precision=lax.Precision.HIGHEST)


def _mlp_mid_body(t_ref, p_ref, b1a2_ref, w1b_ref, b1b_ref, w2a_ref, o_ref):
    w_mid = jnp.dot(w1b_ref[...], w2a_ref[...],
                    preferred_element_type=jnp.float32,
                    precision=lax.Precision.HIGHEST)
    b_mid = jnp.dot(b1b_ref[...], w2a_ref[...],
                    preferred_element_type=jnp.float32,
                    precision=lax.Precision.HIGHEST)
    b_mid2 = jnp.concatenate([b_mid, b_mid], axis=1)
    a = jnp.maximum(t_ref[...] + p_ref[0] + p_ref[1] + b1a2_ref[...], 0.0)
    o_ref[...] = jnp.dot(a, _blockdiag(w_mid),
                         preferred_element_type=jnp.float32,
                         precision=lax.Precision.HIGHEST) + b_mid2


def _mlp_out_body(u_ref, q_ref, b2a2_ref, w2b_ref, b2b_ref, wo_ref, bo_ref,
                  o_ref):
    h = w2b_ref.shape[0]
    wv = jnp.dot(w2b_ref[...], wo_ref[...], preferred_element_type=jnp.float32,
                 precision=lax.Precision.HIGHEST)
    bv = jnp.dot(b2b_ref[...], wo_ref[...], preferred_element_type=jnp.float32,
                 precision=lax.Precision.HIGHEST) + bo_ref[...]
    a = jnp.maximum(u_ref[...] + q_ref[0] + q_ref[1] + b2a2_ref[...], 0.0)
    s0 = jnp.sum(a[:, :h] * wv[:, 0], axis=1, keepdims=True)
    s1 = jnp.sum(a[:, h:] * wv[:, 0], axis=1, keepdims=True)
    o_ref[...] = jnp.concatenate([s0, s1], axis=1) + bv


_NU = 2


@functools.lru_cache(maxsize=None)
def _make_sc_agg(np_, h, nchw):
    rpt = np_ // _NS
    ng = nchw // _NU
    nzc = rpt // _CH
    assert nchw % _NU == 0
    mesh = plsc.VectorSubcoreMesh(core_axis_name="c", subcore_axis_name="s")

    @functools.partial(
        pl.kernel,
        out_type=jax.ShapeDtypeStruct((_NC, np_, h), jnp.float32),
        mesh=mesh,
        scratch_types=[
            pltpu.VMEM((nchw, _CH), jnp.int32),
            pltpu.VMEM((nchw, _CH), jnp.int32),
            pltpu.VMEM((_NU, _CH, h), jnp.float32),
            pltpu.VMEM_SHARED((np_, h), jnp.float32),
            pltpu.VMEM_SHARED((np_, h), jnp.float32),
            [pltpu.SemaphoreType.DMA] * _NU,
            [pltpu.SemaphoreType.DMA] * _NU,
        ],
        compiler_params=pltpu.CompilerParams(use_tc_tiling_on_sc=False),
    )
    def agg(t_hbm, edges_hbm, zero_hbm, out_hbm, src_v, dst_v, rows_v,
            acc_sh, t_sh, sg, ss):
        c = lax.axis_index("c")
        s = lax.axis_index("s")
        wid = c * _NS + s
        base = s * rpt
        pltpu.sync_copy(t_hbm.at[pl.ds(base, rpt)], t_sh.at[pl.ds(base, rpt)])
        pltpu.sync_copy(zero_hbm, rows_v.at[0])
        for k in range(nzc):
            pltpu.sync_copy(rows_v.at[0], acc_sh.at[pl.ds(base + k * _CH, _CH)])
        pltpu.sync_copy(edges_hbm.at[0, wid], src_v)
        pltpu.sync_copy(edges_hbm.at[1, wid], dst_v)
        plsc.subcore_barrier()

        def group(g, carry):
            j0 = g * _NU
            gat = [pltpu.async_copy(t_sh.at[src_v.at[j0 + k]], rows_v.at[k],
                                    sg[k]) for k in range(_NU)]
            sca = []
            for k in range(_NU):
                gat[k].wait()
                sca.append(pltpu.async_copy(rows_v.at[k],
                                            acc_sh.at[dst_v.at[j0 + k]],
                                            ss[k], add=True))
            for d in sca:
                d.wait()
            return carry

        lax.fori_loop(0, ng, group, 0)
        plsc.subcore_barrier()
        for k in range(nzc):
            pltpu.sync_copy(acc_sh.at[pl.ds(base + k * _CH, _CH)], rows_v.at[0])
            pltpu.sync_copy(rows_v.at[0], out_hbm.at[c, pl.ds(base + k * _CH, _CH)])

    return agg


def _blk(shp):
    return pl.BlockSpec(shp, lambda i: (i, 0))


def _whole(shp):
    return pl.BlockSpec(shp, lambda i: (0, 0))


def kernel(x, edge_index, W1a, b1a, W1b, b1b, W2a, b2a, W2b, b2b, Wo, bo):
    n, d = x.shape
    h = W1a.shape[1]
    e = edge_index.shape[1]
    np_ = ((n + 1 + _ROW_BLK - 1) // _ROW_BLK) * _ROW_BLK
    nblk = np_ // _ROW_BLK
    nchw = -(-e // (_NW * _CH * _NU)) * _NU
    e_pad = _NW * _CH * nchw

    npk = np_ // 2
    w2h = 2 * h
    pblk2 = npk // nblk

    pad_n = e_pad - e
    fill_src = jnp.full((pad_n,), n, jnp.int32)
    fill_dst = n + 1 + jnp.arange(pad_n, dtype=jnp.int32) % (np_ - n - 1)
    edges = jnp.concatenate([edge_index, jnp.stack([fill_src, fill_dst])],
                            axis=1).reshape(2, _NW, nchw, _CH)
    x2 = jnp.pad(x, ((0, np_ - n), (0, 0))).reshape(npk, 2 * d)
    zeros = jnp.zeros((_CH, h), jnp.float32)

    b1a2, b2a2 = (jnp.concatenate([v, v]).reshape(1, w2h) for v in (b1a, b2a))
    b1b_r, b2b_r = b1b.reshape(1, h), b2b.reshape(1, h)
    bo_r = bo.reshape(1, 1)

    t2 = pl.pallas_call(
        _matmul_body,
        grid=(nblk,),
        in_specs=[_blk((pblk2, 2 * d)), _whole((d, h))],
        out_specs=_blk((pblk2, w2h)),
        out_shape=jax.ShapeDtypeStruct((npk, w2h), jnp.float32),
    )(x2, W1a)

    sc_agg = _make_sc_agg(np_, h, nchw)

    pspec = pl.BlockSpec((2, pblk2, w2h), lambda i: (0, i, 0))

    p = sc_agg(t2.reshape(np_, h), edges, zeros)
    p2 = p.reshape(2, npk, w2h)

    u2 = pl.pallas_call(
        _mlp_mid_body,
        grid=(nblk,),
        in_specs=[_blk((pblk2, w2h)), pspec,
                  _whole((1, w2h)), _whole((h, h)), _whole((1, h)),
                  _whole((h, h))],
        out_specs=_blk((pblk2, w2h)),
        out_shape=jax.ShapeDtypeStruct((npk, w2h), jnp.float32),
    )(t2, p2, b1a2, W1b, b1b_r, W2a)

    q = sc_agg(u2.reshape(np_, h), edges, zeros)
    q2 = q.reshape(2, npk, w2h)

    res = pl.pallas_call(
        _mlp_out_body,
        grid=(nblk,),
        in_specs=[_blk((pblk2, w2h)), pspec,
                  _whole((1, w2h)), _whole((h, h)), _whole((1, h)),
                  _whole((h, 1)), _whole((1, 1))],
        out_specs=_blk((pblk2, 2)),
        out_shape=jax.ShapeDtypeStruct((npk, 2), jnp.float32),
    )(u2, q2, b2a2, W2b, b2b_r, Wo, bo_r)

    return res.reshape(np_)[:n]


# speedup vs baseline: 1.2639x; 1.0279x over previous
import functools

import jax
import jax.numpy as jnp
from jax import lax
from jax.experimental import pallas as pl
from jax.experimental.pallas import tpu as pltpu
from jax.experimental.pallas import tpu_sc as plsc

_NC = 2
_NS = 16
_NW = _NC * _NS
_CH = 128
_ROW_BLK = 2048


def _blockdiag(w):
    h_in, h_out = w.shape
    z = jnp.zeros((h_in, h_out), jnp.float32)
    top = jnp.concatenate([w, z], axis=1)
    bot = jnp.concatenate([z, w], axis=1)
    return jnp.concatenate([top, bot], axis=0)


def _matmul_body(x2_ref, w_ref, o_ref):
    w2 = _blockdiag(w_ref[...])
    o_ref[...] = jnp.dot(x2_ref[...], w2, preferred_element_type=jnp.float32,
                         precision=lax.Precision.HIGHEST)


def _mlp_mid_body(t_ref, p_ref, b1a2_ref, w1b_ref, b1b_ref, w2a_ref, o_ref):
    w_mid = jnp.dot(w1b_ref[...], w2a_ref[...],
                    preferred_element_type=jnp.float32,
                    precision=lax.Precision.HIGHEST)
    b_mid = jnp.dot(b1b_ref[...], w2a_ref[...],
                    preferred_element_type=jnp.float32,
                    precision=lax.Precision.HIGHEST)
    b_mid2 = jnp.concatenate([b_mid, b_mid], axis=1)
    a = jnp.maximum(t_ref[...] + p_ref[0] + p_ref[1] + b1a2_ref[...], 0.0)
    o_ref[...] = jnp.dot(a, _blockdiag(w_mid),
                         preferred_element_type=jnp.float32,
                         precision=lax.Precision.HIGHEST) + b_mid2


def _mlp_out_body(u_ref, q_ref, b2a2_ref, w2b_ref, b2b_ref, wo_ref, bo_ref,
                  o_ref):
    h = w2b_ref.shape[0]
    wv = jnp.dot(w2b_ref[...], wo_ref[...], preferred_element_type=jnp.float32,
                 precision=lax.Precision.HIGHEST)
    bv = jnp.dot(b2b_ref[...], wo_ref[...], preferred_element_type=jnp.float32,
                 precision=lax.Precision.HIGHEST) + bo_ref[...]
    a = jnp.maximum(u_ref[...] + q_ref[0] + q_ref[1] + b2a2_ref[...], 0.0)
    s0 = jnp.sum(a[:, :h] * wv[:, 0], axis=1, keepdims=True)
    s1 = jnp.sum(a[:, h:] * wv[:, 0], axis=1, keepdims=True)
    o_ref[...] = jnp.concatenate([s0, s1], axis=1) + bv


_NU = 2


@functools.lru_cache(maxsize=None)
def _make_sc_agg(np_, h, nchw):
    rpt = np_ // _NS
    ng = nchw // _NU
    nzc = rpt // _CH
    assert nchw % _NU == 0
    mesh = plsc.VectorSubcoreMesh(core_axis_name="c", subcore_axis_name="s")

    @functools.partial(
        pl.kernel,
        out_type=jax.ShapeDtypeStruct((_NC, np_, h), jnp.float32),
        mesh=mesh,
        scratch_types=[
            pltpu.VMEM((nchw, _CH), jnp.int32),
            pltpu.VMEM((nchw, _CH), jnp.int32),
            pltpu.VMEM((_NU, _CH, h), jnp.float32),
            pltpu.VMEM_SHARED((np_, h), jnp.float32),
            pltpu.VMEM_SHARED((np_, h), jnp.float32),
            [pltpu.SemaphoreType.DMA] * _NU,
            [pltpu.SemaphoreType.DMA] * _NU,
        ],
        compiler_params=pltpu.CompilerParams(use_tc_tiling_on_sc=False),
    )
    def agg(t_hbm, edges_hbm, zero_hbm, out_hbm, src_v, dst_v, rows_v,
            acc_sh, t_sh, sg, ss):
        c = lax.axis_index("c")
        s = lax.axis_index("s")
        wid = c * _NS + s
        base = s * rpt
        pltpu.sync_copy(t_hbm.at[pl.ds(base, rpt)], t_sh.at[pl.ds(base, rpt)])
        pltpu.sync_copy(zero_hbm, rows_v.at[0])
        for k in range(nzc):
            pltpu.sync_copy(rows_v.at[0], acc_sh.at[pl.ds(base + k * _CH, _CH)])
        pltpu.sync_copy(edges_hbm.at[0, wid], src_v)
        pltpu.sync_copy(edges_hbm.at[1, wid], dst_v)
        plsc.subcore_barrier()

        def group(g, carry):
            j0 = g * _NU
            gat = [pltpu.async_copy(t_sh.at[src_v.at[j0 + k]], rows_v.at[k],
                                    sg[k]) for k in range(_NU)]
            sca = []
            for k in range(_NU):
                gat[k].wait()
                sca.append(pltpu.async_copy(rows_v.at[k],
                                            acc_sh.at[dst_v.at[j0 + k]],
                                            ss[k], add=True))
            for d in sca:
                d.wait()
            return carry

        lax.fori_loop(0, ng, group, 0)
        plsc.subcore_barrier()
        for k in range(nzc):
            pltpu.sync_copy(acc_sh.at[pl.ds(base + k * _CH, _CH)], rows_v.at[0])
            pltpu.sync_copy(rows_v.at[0], out_hbm.at[c, pl.ds(base + k * _CH, _CH)])

    return agg


def _blk(shp):
    return pl.BlockSpec(shp, lambda i: (i, 0))


def _whole(shp):
    return pl.BlockSpec(shp, lambda i: (0, 0))


def kernel(x, edge_index, W1a, b1a, W1b, b1b, W2a, b2a, W2b, b2b, Wo, bo):
    n, d = x.shape
    h = W1a.shape[1]
    e = edge_index.shape[1]
    np_ = ((n + 1 + _ROW_BLK - 1) // _ROW_BLK) * _ROW_BLK
    nblk = np_ // _ROW_BLK
    nchw = -(-e // (_NW * _CH * _NU)) * _NU
    e_pad = _NW * _CH * nchw

    npk = np_ // 2
    w2h = 2 * h
    pblk2 = npk // nblk

    pad_n = e_pad - e
    fill_src = jnp.full((pad_n,), n, jnp.int32)
    fill_dst = n + 1 + jnp.arange(pad_n, dtype=jnp.int32) % (np_ - n - 1)
    edges = jnp.concatenate([edge_index, jnp.stack([fill_src, fill_dst])],
                            axis=1).reshape(2, _NW, nchw, _CH)
    x2 = jnp.pad(x, ((0, np_ - n), (0, 0))).reshape(npk, 2 * d)
    zeros = jnp.zeros((_CH, h), jnp.float32)

    b1a2, b2a2 = (jnp.concatenate([v, v]).reshape(1, w2h) for v in (b1a, b2a))
    b1b_r, b2b_r = b1b.reshape(1, h), b2b.reshape(1, h)
    bo_r = bo.reshape(1, 1)

    t2 = pl.pallas_call(
        _matmul_body,
        grid=(nblk,),
        in_specs=[_blk((pblk2, 2 * d)), _whole((d, h))],
        out_specs=_blk((pblk2, w2h)),
        out_shape=jax.ShapeDtypeStruct((npk, w2h), jnp.float32),
    )(x2, W1a)

    sc_agg = _make_sc_agg(np_, h, nchw)

    pspec = pl.BlockSpec((2, pblk2, w2h), lambda i: (0, i, 0))

    p = sc_agg(t2.reshape(np_, h), edges, zeros)
    p2 = p.reshape(2, npk, w2h)

    u2 = pl.pallas_call(
        _mlp_mid_body,
        grid=(nblk,),
        in_specs=[_blk((pblk2, w2h)), pspec,
                  _whole((1, w2h)), _whole((h, h)), _whole((1, h)),
                  _whole((h, h))],
        out_specs=_blk((pblk2, w2h)),
        out_shape=jax.ShapeDtypeStruct((npk, w2h), jnp.float32),
    )(t2, p2, b1a2, W1b, b1b_r, W2a)

    q = sc_agg(u2.reshape(np_, h), edges, zeros)
    q2 = q.reshape(2, npk, w2h)

    res = pl.pallas_call(
        _mlp_out_body,
        grid=(nblk,),
        in_specs=[_blk((pblk2, w2h)), pspec,
                  _whole((1, w2h)), _whole((h, h)), _whole((1, h)),
                  _whole((h, 1)), _whole((1, 1))],
        out_specs=_blk((pblk2, 2)),
        out_shape=jax.ShapeDtypeStruct((npk, 2), jnp.float32),
    )(u2, q2, b2a2, W2b, b2b_r, Wo, bo_r)

    return res.reshape(np_)[:n]


# speedup vs baseline: 1.2889x; 1.0198x over previous
import functools

import jax
import jax.numpy as jnp
from jax import lax
from jax.experimental import pallas as pl
from jax.experimental.pallas import tpu as pltpu
from jax.experimental.pallas import tpu_sc as plsc

_NC = 2
_NS = 16
_NW = _NC * _NS
_CH = 128
_ROW_BLK = 5120


def _blockdiag(w):
    h_in, h_out = w.shape
    z = jnp.zeros((h_in, h_out), jnp.float32)
    top = jnp.concatenate([w, z], axis=1)
    bot = jnp.concatenate([z, w], axis=1)
    return jnp.concatenate([top, bot], axis=0)


def _matmul_body(x2_ref, w_ref, o_ref):
    w2 = _blockdiag(w_ref[...])
    o_ref[...] = jnp.dot(x2_ref[...], w2, preferred_element_type=jnp.float32,
                         precision=lax.Precision.HIGHEST)


def _mlp_mid_body(t_ref, p_ref, b1a2_ref, w1b_ref, b1b_ref, w2a_ref, o_ref):
    w_mid = jnp.dot(w1b_ref[...], w2a_ref[...],
                    preferred_element_type=jnp.float32,
                    precision=lax.Precision.HIGHEST)
    b_mid = jnp.dot(b1b_ref[...], w2a_ref[...],
                    preferred_element_type=jnp.float32,
                    precision=lax.Precision.HIGHEST)
    b_mid2 = jnp.concatenate([b_mid, b_mid], axis=1)
    a = jnp.maximum(t_ref[...] + p_ref[0] + p_ref[1] + b1a2_ref[...], 0.0)
    o_ref[...] = jnp.dot(a, _blockdiag(w_mid),
                         preferred_element_type=jnp.float32,
                         precision=lax.Precision.HIGHEST) + b_mid2


def _mlp_out_body(u_ref, q_ref, b2a2_ref, w2b_ref, b2b_ref, wo_ref, bo_ref,
                  o_ref):
    h = w2b_ref.shape[0]
    wv = jnp.dot(w2b_ref[...], wo_ref[...], preferred_element_type=jnp.float32,
                 precision=lax.Precision.HIGHEST)
    bv = jnp.dot(b2b_ref[...], wo_ref[...], preferred_element_type=jnp.float32,
                 precision=lax.Precision.HIGHEST) + bo_ref[...]
    a = jnp.maximum(u_ref[...] + q_ref[0] + q_ref[1] + b2a2_ref[...], 0.0)
    s0 = jnp.sum(a[:, :h] * wv[:, 0], axis=1, keepdims=True)
    s1 = jnp.sum(a[:, h:] * wv[:, 0], axis=1, keepdims=True)
    o_ref[...] = jnp.concatenate([s0, s1], axis=1) + bv


_NU = 2


@functools.lru_cache(maxsize=None)
def _make_sc_agg(np_, h, nchw):
    rpt = np_ // _NS
    ng = nchw // _NU
    nzc = rpt // _CH
    assert nchw % _NU == 0
    mesh = plsc.VectorSubcoreMesh(core_axis_name="c", subcore_axis_name="s")

    @functools.partial(
        pl.kernel,
        out_type=jax.ShapeDtypeStruct((_NC, np_, h), jnp.float32),
        mesh=mesh,
        scratch_types=[
            pltpu.VMEM((nchw, _CH), jnp.int32),
            pltpu.VMEM((nchw, _CH), jnp.int32),
            pltpu.VMEM((_NU, _CH, h), jnp.float32),
            pltpu.VMEM_SHARED((np_, h), jnp.float32),
            pltpu.VMEM_SHARED((np_, h), jnp.float32),
            [pltpu.SemaphoreType.DMA] * _NU,
            [pltpu.SemaphoreType.DMA] * _NU,
        ],
        compiler_params=pltpu.CompilerParams(use_tc_tiling_on_sc=False),
    )
    def agg(t_hbm, edges_hbm, zero_hbm, out_hbm, src_v, dst_v, rows_v,
            acc_sh, t_sh, sg, ss):
        c = lax.axis_index("c")
        s = lax.axis_index("s")
        wid = c * _NS + s
        base = s * rpt
        pltpu.sync_copy(t_hbm.at[pl.ds(base, rpt)], t_sh.at[pl.ds(base, rpt)])
        pltpu.sync_copy(zero_hbm, rows_v.at[0])
        for k in range(nzc):
            pltpu.sync_copy(rows_v.at[0], acc_sh.at[pl.ds(base + k * _CH, _CH)])
        pltpu.sync_copy(edges_hbm.at[0, wid], src_v)
        pltpu.sync_copy(edges_hbm.at[1, wid], dst_v)
        plsc.subcore_barrier()

        def group(g, carry):
            j0 = g * _NU
            gat = [pltpu.async_copy(t_sh.at[src_v.at[j0 + k]], rows_v.at[k],
                                    sg[k]) for k in range(_NU)]
            sca = []
            for k in range(_NU):
                gat[k].wait()
                sca.append(pltpu.async_copy(rows_v.at[k],
                                            acc_sh.at[dst_v.at[j0 + k]],
                                            ss[k], add=True))
            for d in sca:
                d.wait()
            return carry

        lax.fori_loop(0, ng, group, 0)
        plsc.subcore_barrier()
        for k in range(nzc):
            pltpu.sync_copy(acc_sh.at[pl.ds(base + k * _CH, _CH)], rows_v.at[0])
            pltpu.sync_copy(rows_v.at[0], out_hbm.at[c, pl.ds(base + k * _CH, _CH)])

    return agg


def _blk(shp):
    return pl.BlockSpec(shp, lambda i: (i, 0))


def _whole(shp):
    return pl.BlockSpec(shp, lambda i: (0, 0))


def kernel(x, edge_index, W1a, b1a, W1b, b1b, W2a, b2a, W2b, b2b, Wo, bo):
    n, d = x.shape
    h = W1a.shape[1]
    e = edge_index.shape[1]
    np_ = ((n + 1 + _ROW_BLK - 1) // _ROW_BLK) * _ROW_BLK
    nblk = np_ // _ROW_BLK
    nchw = -(-e // (_NW * _CH * _NU)) * _NU
    e_pad = _NW * _CH * nchw

    npk = np_ // 2
    w2h = 2 * h
    pblk2 = npk // nblk

    pad_n = e_pad - e
    fill_src = jnp.full((pad_n,), n, jnp.int32)
    fill_dst = n + 1 + jnp.arange(pad_n, dtype=jnp.int32) % (np_ - n - 1)
    edges = jnp.concatenate([edge_index, jnp.stack([fill_src, fill_dst])],
                            axis=1).reshape(2, _NW, nchw, _CH)
    x2 = jnp.pad(x, ((0, np_ - n), (0, 0))).reshape(npk, 2 * d)
    zeros = jnp.zeros((_CH, h), jnp.float32)

    b1a2, b2a2 = (jnp.concatenate([v, v]).reshape(1, w2h) for v in (b1a, b2a))
    b1b_r, b2b_r = b1b.reshape(1, h), b2b.reshape(1, h)
    bo_r = bo.reshape(1, 1)

    t2 = pl.pallas_call(
        _matmul_body,
        grid=(nblk,),
        in_specs=[_blk((pblk2, 2 * d)), _whole((d, h))],
        out_specs=_blk((pblk2, w2h)),
        out_shape=jax.ShapeDtypeStruct((npk, w2h), jnp.float32),
    )(x2, W1a)

    sc_agg = _make_sc_agg(np_, h, nchw)

    pspec = pl.BlockSpec((2, pblk2, w2h), lambda i: (0, i, 0))

    p = sc_agg(t2.reshape(np_, h), edges, zeros)
    p2 = p.reshape(2, npk, w2h)

    u2 = pl.pallas_call(
        _mlp_mid_body,
        grid=(nblk,),
        in_specs=[_blk((pblk2, w2h)), pspec,
                  _whole((1, w2h)), _whole((h, h)), _whole((1, h)),
                  _whole((h, h))],
        out_specs=_blk((pblk2, w2h)),
        out_shape=jax.ShapeDtypeStruct((npk, w2h), jnp.float32),
    )(t2, p2, b1a2, W1b, b1b_r, W2a)

    q = sc_agg(u2.reshape(np_, h), edges, zeros)
    q2 = q.reshape(2, npk, w2h)

    res = pl.pallas_call(
        _mlp_out_body,
        grid=(nblk,),
        in_specs=[_blk((pblk2, w2h)), pspec,
                  _whole((1, w2h)), _whole((h, h)), _whole((1, h)),
                  _whole((h, 1)), _whole((1, 1))],
        out_specs=_blk((pblk2, 2)),
        out_shape=jax.ShapeDtypeStruct((npk, 2), jnp.float32),
    )(u2, q2, b2a2, W2b, b2b_r, Wo, bo_r)

    return res.reshape(np_)[:n]
